# baseline (speedup 1.0000x reference)
import functools

import jax
import jax.numpy as jnp
from jax import lax
from jax.experimental import pallas as pl
from jax.experimental.pallas import tpu as pltpu
from jax.experimental.pallas import tpu_sc as plsc

B, C, V = 64, 4, 10000
DEG = 16
CL1_F, CL1_K = 32, 3
CL2_F, CL2_K = 64, 3
FC1_F, FC2_F = 512, 10
V2 = V // 4
FC1_IN = CL2_F * V // 16

NC, NS = 2, 16

FC_OCHUNK = 64
FC_STEPS = FC1_F // FC_OCHUNK


def _fc_body(h_ref, w1_ref, b1_ref, w2_ref, b2_ref, out_ref):
    k = pl.program_id(0)

    @pl.when(k == 0)
    def _init():
        out_ref[...] = jnp.broadcast_to(b2_ref[...], out_ref.shape)

    z = jax.nn.sigmoid(
        jax.lax.dot_general(h_ref[...], w1_ref[...], (((1,), (1,)), ((), ())),
                            preferred_element_type=jnp.float32)
        + b1_ref[0])
    out_ref[...] += jax.lax.dot_general(
        z, w2_ref[0], (((1,), (1,)), ((), ())),
        preferred_element_type=jnp.float32)


def _fc_head(h, FC1_w, FC1_b, FC2_w, FC2_b):
    return pl.pallas_call(
        _fc_body,
        grid=(FC_STEPS,),
        in_specs=[
            pl.BlockSpec((B, FC1_IN), lambda k: (0, 0)),
            pl.BlockSpec((FC_OCHUNK, FC1_IN), lambda k: (k, 0)),
            pl.BlockSpec((1, 1, FC_OCHUNK), lambda k: (k, 0, 0)),
            pl.BlockSpec((1, FC2_F, FC_OCHUNK), lambda k: (k, 0, 0)),
            pl.BlockSpec((1, FC2_F), lambda k: (0, 0)),
        ],
        out_specs=pl.BlockSpec((B, FC2_F), lambda k: (0, 0)),
        out_shape=jax.ShapeDtypeStruct((B, FC2_F), jnp.float32),
    )(h, FC1_w,
      FC1_b.reshape(FC_STEPS, 1, FC_OCHUNK),
      FC2_w.reshape(FC2_F, FC_STEPS, FC_OCHUNK).transpose(1, 0, 2),
      FC2_b.reshape(1, FC2_F))


@functools.lru_cache(maxsize=None)
def _make_spmm(Vpad, Wc, NCHUNK, E_pad, KB):
    KC = NCHUNK // NC
    RPT = Vpad // NS
    assert RPT * NS == Vpad and RPT % 8 == 0 and E_pad % KB == 0
    EPK = E_pad + KB
    mesh = plsc.VectorSubcoreMesh(core_axis_name="c", subcore_axis_name="s")

    @functools.partial(
        pl.kernel, mesh=mesh,
        out_type=jax.ShapeDtypeStruct((NCHUNK * Vpad, Wc), jnp.float32),
        scratch_types=[
            pltpu.VMEM((16,), jnp.int32),
            pltpu.VMEM((KB,), jnp.int32),
            pltpu.VMEM((KB,), jnp.int32),
            pltpu.VMEM((KB * 16,), jnp.float32),
            pltpu.VMEM((KB, Wc), jnp.float32),
            pltpu.VMEM((RPT, Wc), jnp.float32),
            pltpu.SemaphoreType.DMA,
        ],
    )
    def spmm(x_hbm, colsadj_hbm, rows_hbm, vals_hbm, out_hbm,
             psm, rsm, colv, valv, xg, acc, sem):
        c = lax.axis_index("c")
        s = lax.axis_index("s")
        pltpu.sync_copy(rows_hbm.at[pl.ds(EPK + s * 8, 16)], psm)
        pv = psm[pl.ds(0, 16)]
        lo = pv[0]
        hi = pv[8]
        e00 = (lo // KB) * KB
        nb = (hi - e00 + KB - 1) // KB
        row0 = s * RPT

        for kc in range(KC):
            q = c * KC + kc

            def zbody(r, carry):
                for j in range(Wc // 16):
                    acc[r, pl.ds(j * 16, 16)] = jnp.zeros((16,), jnp.float32)
                return carry

            lax.fori_loop(0, RPT, zbody, 0)

            def batch(i, carry):
                e0 = e00 + i * KB
                pltpu.sync_copy(colsadj_hbm.at[pl.ds(q * EPK + e0, KB)], colv)
                pltpu.sync_copy(rows_hbm.at[pl.ds(e0, KB)], rsm)
                pltpu.sync_copy(vals_hbm.at[pl.ds(e0 * 16, KB * 16)], valv)
                pltpu.async_copy(x_hbm.at[colv], xg, sem).wait()

                def group(g, carry2):
                    rv = rsm[pl.ds(g * 16, 16)]
                    for ri in range(16):
                        r = g * 16 + ri
                        e = e0 + r
                        valid = jnp.logical_and(e >= lo, e < hi)
                        vm = jnp.where(valid, jnp.float32(1.0),
                                       jnp.float32(0.0))
                        rl = jnp.clip(rv[ri] - row0, 0, RPT - 1)
                        vv = valv[pl.ds(g * 256 + ri * 16, 16)] * vm
                        for j in range(Wc // 16):
                            sl = (rl, pl.ds(j * 16, 16))
                            acc[sl] = acc[sl] + xg[r, pl.ds(j * 16, 16)] * vv
                    return carry2

                lax.fori_loop(0, KB // 16, group, 0)
                return carry

            lax.fori_loop(0, nb, batch, 0)
            pltpu.sync_copy(acc, out_hbm.at[pl.ds(q * Vpad + s * RPT, RPT)])

    return spmm


def _stack(x, NCHUNK, Wc, Vpad):
    Vv = x.shape[0]
    xs = x.reshape(Vv, NCHUNK, Wc).transpose(1, 0, 2)
    if Vpad > Vv:
        xs = jnp.pad(xs, ((0, 0), (0, Vpad - Vv), (0, 0)))
    return xs.reshape(NCHUNK * Vpad, Wc)


def _unstack(y, NCHUNK, Wc, Vpad, Vv):
    return (y.reshape(NCHUNK, Vpad, Wc)[:, :Vv]
            .transpose(1, 0, 2).reshape(Vv, NCHUNK * Wc))


def _prep_edges(rows, cols, vals, E_pad, KB, NCHUNK, Vpad):
    E = rows.shape[0]
    RPT = Vpad // NS
    order = jnp.argsort(rows)
    rows_s = rows[order].astype(jnp.int32)
    cols_s = cols[order].astype(jnp.int32)
    vals_s = vals[order]
    pad = E_pad - E
    rows_p = jnp.concatenate(
        [rows_s, jnp.full((pad,), Vpad - 1, jnp.int32)]) if pad else rows_s
    cols_p = jnp.pad(cols_s, (0, pad))
    vals_p = jnp.pad(vals_s, (0, pad))
    bounds = jnp.arange(NS + 1, dtype=jnp.int32) * RPT
    ptr = jnp.searchsorted(rows_p, bounds, side='left').astype(jnp.int32)
    ptr = ptr.at[NS].set(E_pad)
    ptr8 = jnp.zeros((NS + 1, 8), jnp.int32).at[:, 0].set(ptr).reshape(-1)
    rows_full = jnp.concatenate(
        [rows_p, jnp.zeros((KB,), jnp.int32), ptr8])
    offs = (jnp.arange(NCHUNK, dtype=jnp.int32) * Vpad)[:, None]
    cols_full = jnp.pad(cols_p, (0, KB))
    colsadj = (cols_full[None, :] + offs).reshape(-1)
    vals_x = jnp.repeat(jnp.pad(vals_p, (0, KB)), 16)
    return colsadj, rows_full, vals_x


def _combine(xs, W, b):
    K = len(xs)
    Vv = xs[0].shape[0]
    X = jnp.stack(xs, axis=0).reshape(K, Vv, -1, B)
    Cc = X.shape[2]
    X = jnp.transpose(X, (3, 1, 2, 0)).reshape(B * Vv, Cc * K)
    out = (X @ W.T + b).reshape(B, Vv, -1)
    return jnp.transpose(out, (0, 2, 1))


def _maxpool4(x):
    Bb, Ff, Vv = x.shape
    return x.reshape(Bb, Ff, Vv // 4, 4).max(axis=-1)


def kernel(x, rows1, cols1, vals1, rows2, cols2, vals2,
           GCL1_w, GCL1_b, GCL2_w, GCL2_b, FC1_w, FC1_b, FC2_w, FC2_b):
    W1, Wc1, NCH1, KB1 = C * B, 128, 2, 128
    VP1 = NS * 8 * -(-V // (NS * 8))
    EP1 = KB1 * -(-160000 // KB1)
    spmm1 = _make_spmm(VP1, Wc1, NCH1, EP1, KB1)

    h = x / jnp.sqrt(1.0 + 1e-5)
    x0 = jnp.transpose(h, (2, 1, 0)).reshape(V, W1)
    ca1, r1, v1 = _prep_edges(rows1, cols1, vals1, EP1, KB1, NCH1, VP1)
    st0 = _stack(x0, NCH1, Wc1, VP1)
    t1 = spmm1(st0, ca1, r1, v1)
    t2 = spmm1(t1, ca1, r1, v1)
    x1 = _unstack(t1, NCH1, Wc1, VP1, V)
    x2 = 2.0 * _unstack(t2, NCH1, Wc1, VP1, V) - x0

    h = _maxpool4(jax.nn.relu(_combine([x0, x1, x2], GCL1_w, GCL1_b)))

    W2, Wc2, NCH2, KB2 = CL1_F * B, 512, 4, 32
    VP2 = NS * 8 * -(-V2 // (NS * 8))
    EP2 = KB2 * -(-40000 // KB2)
    spmm2 = _make_spmm(VP2, Wc2, NCH2, EP2, KB2)

    x0b = jnp.transpose(h, (2, 1, 0)).reshape(V2, W2)
    ca2, r2, v2 = _prep_edges(rows2, cols2, vals2, EP2, KB2, NCH2, VP2)
    st0b = _stack(x0b, NCH2, Wc2, VP2)
    u1 = spmm2(st0b, ca2, r2, v2)
    u2 = spmm2(u1, ca2, r2, v2)
    x1b = _unstack(u1, NCH2, Wc2, VP2, V2)
    x2b = 2.0 * _unstack(u2, NCH2, Wc2, VP2, V2) - x0b

    h = _maxpool4(jax.nn.relu(_combine([x0b, x1b, x2b], GCL2_w, GCL2_b)))
    h = h.reshape(B, -1)
    return _fc_head(h, FC1_w, FC1_b, FC2_w, FC2_b)


# speedup vs baseline: 1.0696x; 1.0696x over previous
import functools

import jax
import jax.numpy as jnp
from jax import lax
from jax.experimental import pallas as pl
from jax.experimental.pallas import tpu as pltpu
from jax.experimental.pallas import tpu_sc as plsc

B, C, V = 64, 4, 10000
DEG = 16
CL1_F, CL1_K = 32, 3
CL2_F, CL2_K = 64, 3
FC1_F, FC2_F = 512, 10
V2 = V // 4
FC1_IN = CL2_F * V // 16

NC, NS = 2, 16

FC_OCHUNK = 64
FC_STEPS = FC1_F // FC_OCHUNK


def _fc_body(h_ref, w1_ref, b1_ref, w2_ref, b2_ref, out_ref):
    k = pl.program_id(0)

    @pl.when(k == 0)
    def _init():
        out_ref[...] = jnp.broadcast_to(b2_ref[...], out_ref.shape)

    z = jax.nn.sigmoid(
        jax.lax.dot_general(h_ref[...], w1_ref[...], (((1,), (1,)), ((), ())),
                            preferred_element_type=jnp.float32)
        + b1_ref[0])
    out_ref[...] += jax.lax.dot_general(
        z, w2_ref[0], (((1,), (1,)), ((), ())),
        preferred_element_type=jnp.float32)


def _fc_head(h, FC1_w, FC1_b, FC2_w, FC2_b):
    return pl.pallas_call(
        _fc_body,
        grid=(FC_STEPS,),
        in_specs=[
            pl.BlockSpec((B, FC1_IN), lambda k: (0, 0)),
            pl.BlockSpec((FC_OCHUNK, FC1_IN), lambda k: (k, 0)),
            pl.BlockSpec((1, 1, FC_OCHUNK), lambda k: (k, 0, 0)),
            pl.BlockSpec((1, FC2_F, FC_OCHUNK), lambda k: (k, 0, 0)),
            pl.BlockSpec((1, FC2_F), lambda k: (0, 0)),
        ],
        out_specs=pl.BlockSpec((B, FC2_F), lambda k: (0, 0)),
        out_shape=jax.ShapeDtypeStruct((B, FC2_F), jnp.float32),
    )(h, FC1_w,
      FC1_b.reshape(FC_STEPS, 1, FC_OCHUNK),
      FC2_w.reshape(FC2_F, FC_STEPS, FC_OCHUNK).transpose(1, 0, 2),
      FC2_b.reshape(1, FC2_F))


@functools.lru_cache(maxsize=None)
def _make_spmm(Vpad, Wc, NCHUNK, E_pad, KB):
    KC = NCHUNK // NC
    RPT = Vpad // NS
    assert RPT * NS == Vpad and RPT % 8 == 0 and E_pad % KB == 0
    EPK = E_pad + KB
    mesh = plsc.VectorSubcoreMesh(core_axis_name="c", subcore_axis_name="s")

    @functools.partial(
        pl.kernel, mesh=mesh,
        out_type=jax.ShapeDtypeStruct((NCHUNK * Vpad, Wc), jnp.float32),
        scratch_types=[
            pltpu.VMEM((16,), jnp.int32),
            pltpu.VMEM((KB,), jnp.int32),
            pltpu.VMEM((KB,), jnp.int32),
            pltpu.VMEM((KB * 16,), jnp.float32),
            pltpu.VMEM((KB, Wc), jnp.float32),
            pltpu.VMEM((RPT, Wc), jnp.float32),
            pltpu.SemaphoreType.DMA,
            pltpu.SemaphoreType.DMA,
        ],
    )
    def spmm(x_hbm, colsadj_hbm, rows_hbm, vals_hbm, out_hbm,
             psm, rsm, colv, valv, xg, acc, sem, sem2):
        c = lax.axis_index("c")
        s = lax.axis_index("s")
        pltpu.sync_copy(rows_hbm.at[pl.ds(EPK + s * 8, 16)], psm)
        pv = psm[pl.ds(0, 16)]
        lo = pv[0]
        hi = pv[8]
        e00 = (lo // KB) * KB
        nb = (hi - e00 + KB - 1) // KB
        row0 = s * RPT

        for kc in range(KC):
            q = c * KC + kc

            def zbody(r, carry):
                for j in range(Wc // 16):
                    acc[r, pl.ds(j * 16, 16)] = jnp.zeros((16,), jnp.float32)
                return carry

            lax.fori_loop(0, RPT, zbody, 0)

            def batch(i, carry):
                e0 = e00 + i * KB
                cpo = pltpu.async_copy(
                    colsadj_hbm.at[pl.ds(q * EPK + e0, KB)], colv, sem2)
                rpo = pltpu.async_copy(rows_hbm.at[pl.ds(e0, KB)], rsm, sem2)
                vpo = pltpu.async_copy(
                    vals_hbm.at[pl.ds(e0 * 16, KB * 16)], valv, sem2)
                cpo.wait()
                rpo.wait()
                vpo.wait()
                pltpu.async_copy(x_hbm.at[colv], xg, sem).wait()

                def group(g, carry2):
                    rv = rsm[pl.ds(g * 16, 16)]
                    for ri in range(16):
                        r = g * 16 + ri
                        e = e0 + r
                        valid = jnp.logical_and(e >= lo, e < hi)
                        vm = jnp.where(valid, jnp.float32(1.0),
                                       jnp.float32(0.0))
                        rl = jnp.clip(rv[ri] - row0, 0, RPT - 1)
                        vv = valv[pl.ds(g * 256 + ri * 16, 16)] * vm
                        for j in range(Wc // 16):
                            sl = (rl, pl.ds(j * 16, 16))
                            acc[sl] = acc[sl] + xg[r, pl.ds(j * 16, 16)] * vv
                    return carry2

                lax.fori_loop(0, KB // 16, group, 0)
                return carry

            lax.fori_loop(0, nb, batch, 0)
            pltpu.sync_copy(acc, out_hbm.at[pl.ds(q * Vpad + s * RPT, RPT)])

    return spmm


def _stack(x, NCHUNK, Wc, Vpad):
    Vv = x.shape[0]
    xs = x.reshape(Vv, NCHUNK, Wc).transpose(1, 0, 2)
    if Vpad > Vv:
        xs = jnp.pad(xs, ((0, 0), (0, Vpad - Vv), (0, 0)))
    return xs.reshape(NCHUNK * Vpad, Wc)


def _unstack(y, NCHUNK, Wc, Vpad, Vv):
    return (y.reshape(NCHUNK, Vpad, Wc)[:, :Vv]
            .transpose(1, 0, 2).reshape(Vv, NCHUNK * Wc))


def _prep_edges(rows, cols, vals, E_pad, KB, NCHUNK, Vpad):
    E = rows.shape[0]
    RPT = Vpad // NS
    order = jnp.argsort(rows)
    rows_s = rows[order].astype(jnp.int32)
    cols_s = cols[order].astype(jnp.int32)
    vals_s = vals[order]
    pad = E_pad - E
    rows_p = jnp.concatenate(
        [rows_s, jnp.full((pad,), Vpad - 1, jnp.int32)]) if pad else rows_s
    cols_p = jnp.pad(cols_s, (0, pad))
    vals_p = jnp.pad(vals_s, (0, pad))
    bounds = jnp.arange(NS + 1, dtype=jnp.int32) * RPT
    ptr = jnp.searchsorted(rows_p, bounds, side='left').astype(jnp.int32)
    ptr = ptr.at[NS].set(E_pad)
    ptr8 = jnp.zeros((NS + 1, 8), jnp.int32).at[:, 0].set(ptr).reshape(-1)
    rows_full = jnp.concatenate(
        [rows_p, jnp.zeros((KB,), jnp.int32), ptr8])
    offs = (jnp.arange(NCHUNK, dtype=jnp.int32) * Vpad)[:, None]
    cols_full = jnp.pad(cols_p, (0, KB))
    colsadj = (cols_full[None, :] + offs).reshape(-1)
    vals_x = jnp.repeat(jnp.pad(vals_p, (0, KB)), 16)
    return colsadj, rows_full, vals_x


def _combine(xs, W, b):
    K = len(xs)
    Vv = xs[0].shape[0]
    X = jnp.stack(xs, axis=0).reshape(K, Vv, -1, B)
    Cc = X.shape[2]
    X = jnp.transpose(X, (3, 1, 2, 0)).reshape(B * Vv, Cc * K)
    out = (X @ W.T + b).reshape(B, Vv, -1)
    return jnp.transpose(out, (0, 2, 1))


def _maxpool4(x):
    Bb, Ff, Vv = x.shape
    return x.reshape(Bb, Ff, Vv // 4, 4).max(axis=-1)


def kernel(x, rows1, cols1, vals1, rows2, cols2, vals2,
           GCL1_w, GCL1_b, GCL2_w, GCL2_b, FC1_w, FC1_b, FC2_w, FC2_b):
    W1, Wc1, NCH1, KB1 = C * B, 128, 2, 128
    VP1 = NS * 8 * -(-V // (NS * 8))
    EP1 = KB1 * -(-160000 // KB1)
    spmm1 = _make_spmm(VP1, Wc1, NCH1, EP1, KB1)

    h = x / jnp.sqrt(1.0 + 1e-5)
    x0 = jnp.transpose(h, (2, 1, 0)).reshape(V, W1)
    ca1, r1, v1 = _prep_edges(rows1, cols1, vals1, EP1, KB1, NCH1, VP1)
    st0 = _stack(x0, NCH1, Wc1, VP1)
    t1 = spmm1(st0, ca1, r1, v1)
    t2 = spmm1(t1, ca1, r1, v1)
    x1 = _unstack(t1, NCH1, Wc1, VP1, V)
    x2 = 2.0 * _unstack(t2, NCH1, Wc1, VP1, V) - x0

    h = _maxpool4(jax.nn.relu(_combine([x0, x1, x2], GCL1_w, GCL1_b)))

    W2, Wc2, NCH2, KB2 = CL1_F * B, 512, 4, 64
    VP2 = NS * 8 * -(-V2 // (NS * 8))
    EP2 = KB2 * -(-40000 // KB2)
    spmm2 = _make_spmm(VP2, Wc2, NCH2, EP2, KB2)

    x0b = jnp.transpose(h, (2, 1, 0)).reshape(V2, W2)
    ca2, r2, v2 = _prep_edges(rows2, cols2, vals2, EP2, KB2, NCH2, VP2)
    st0b = _stack(x0b, NCH2, Wc2, VP2)
    u1 = spmm2(st0b, ca2, r2, v2)
    u2 = spmm2(u1, ca2, r2, v2)
    x1b = _unstack(u1, NCH2, Wc2, VP2, V2)
    x2b = 2.0 * _unstack(u2, NCH2, Wc2, VP2, V2) - x0b

    h = _maxpool4(jax.nn.relu(_combine([x0b, x1b, x2b], GCL2_w, GCL2_b)))
    h = h.reshape(B, -1)
    return _fc_head(h, FC1_w, FC1_b, FC2_w, FC2_b)
